# baseline (speedup 1.0000x reference)
import jax
import jax.numpy as jnp
from jax.experimental import pallas as pl
from jax.experimental.pallas import tpu as pltpu

NCHUNK = 16
ROWS = 16384 // NCHUNK


def _mlp_kernel(x_hbm, wp_ref, bp_ref, w1_ref, b1_ref, w2_ref, b2_ref,
                o_ref, xbuf, we_ref, be_ref, sems):
    for c in range(NCHUNK):
        pltpu.make_async_copy(
            x_hbm.at[pl.ds(c * ROWS, ROWS), :],
            xbuf.at[c],
            sems.at[c],
        ).start()

    w1 = w1_ref[...]
    we = jnp.dot(wp_ref[...], w1, preferred_element_type=jnp.float32)
    we_ref[...] = we.astype(jnp.bfloat16)
    be_ref[...] = (
        jnp.dot(bp_ref[...], w1, preferred_element_type=jnp.float32)
        + b1_ref[...]
    )
    w2 = w2_ref[...].astype(jnp.bfloat16)
    b2 = b2_ref[...]

    def body(c, _):
        pltpu.make_async_copy(
            x_hbm.at[pl.ds(c * ROWS, ROWS), :],
            xbuf.at[c],
            sems.at[c],
        ).wait()
        x = xbuf[c].astype(jnp.bfloat16)
        h = jnp.dot(x, we_ref[...], preferred_element_type=jnp.float32) + be_ref[...]
        h = jnp.maximum(h, 0.0).astype(jnp.bfloat16)
        o_ref[pl.ds(c * ROWS, ROWS), :] = (
            jnp.dot(h, w2, preferred_element_type=jnp.float32) + b2
        )
        return _

    jax.lax.fori_loop(0, NCHUNK, body, 0)


def kernel(trial_feats, Wp, bp, W1, b1, W2, b2):
    B, F = trial_feats.shape
    H = Wp.shape[1]
    O = W2.shape[1]
    return pl.pallas_call(
        _mlp_kernel,
        in_specs=[
            pl.BlockSpec(memory_space=pl.ANY),
            pl.BlockSpec(memory_space=pltpu.MemorySpace.VMEM),
            pl.BlockSpec(memory_space=pltpu.MemorySpace.VMEM),
            pl.BlockSpec(memory_space=pltpu.MemorySpace.VMEM),
            pl.BlockSpec(memory_space=pltpu.MemorySpace.VMEM),
            pl.BlockSpec(memory_space=pltpu.MemorySpace.VMEM),
            pl.BlockSpec(memory_space=pltpu.MemorySpace.VMEM),
        ],
        out_specs=pl.BlockSpec(memory_space=pltpu.MemorySpace.VMEM),
        out_shape=jax.ShapeDtypeStruct((B, O), jnp.float32),
        scratch_shapes=[
            pltpu.VMEM((NCHUNK, ROWS, F), jnp.float32),
            pltpu.VMEM((F, H), jnp.bfloat16),
            pltpu.VMEM((1, H), jnp.float32),
            pltpu.SemaphoreType.DMA((NCHUNK,)),
        ],
    )(trial_feats, Wp, bp.reshape(1, H), W1, b1.reshape(1, H),
      W2, b2.reshape(1, O))


# speedup vs baseline: 1.1221x; 1.1221x over previous
import jax
import jax.numpy as jnp
from jax.experimental import pallas as pl
from jax.experimental.pallas import tpu as pltpu

TILE = 4096


def _mlp_kernel(x_ref, wp_ref, bp_ref, w1_ref, b1_ref, w2_ref, b2_ref,
                o_ref, we_ref, be_ref):
    @pl.when(pl.program_id(0) == 0)
    def _fold():
        w1 = w1_ref[...]
        we_ref[...] = jnp.dot(wp_ref[...], w1, preferred_element_type=jnp.float32)
        be_ref[...] = (
            jnp.dot(bp_ref[...], w1, preferred_element_type=jnp.float32)
            + b1_ref[...]
        )

    h = jnp.dot(x_ref[...], we_ref[...],
                preferred_element_type=jnp.float32) + be_ref[...]
    h = jnp.maximum(h, 0.0)
    o_ref[...] = jnp.dot(h, w2_ref[...],
                         preferred_element_type=jnp.float32) + b2_ref[...]


def kernel(trial_feats, Wp, bp, W1, b1, W2, b2):
    B, F = trial_feats.shape
    H = Wp.shape[1]
    O = W2.shape[1]
    grid = (B // TILE,)
    return pl.pallas_call(
        _mlp_kernel,
        grid=grid,
        in_specs=[
            pl.BlockSpec((TILE, F), lambda i: (i, 0)),
            pl.BlockSpec((F, H), lambda i: (0, 0)),
            pl.BlockSpec((1, H), lambda i: (0, 0)),
            pl.BlockSpec((H, H), lambda i: (0, 0)),
            pl.BlockSpec((1, H), lambda i: (0, 0)),
            pl.BlockSpec((H, O), lambda i: (0, 0)),
            pl.BlockSpec((1, O), lambda i: (0, 0)),
        ],
        out_specs=pl.BlockSpec((TILE, O), lambda i: (i, 0)),
        out_shape=jax.ShapeDtypeStruct((B, O), jnp.float32),
        scratch_shapes=[
            pltpu.VMEM((F, H), jnp.float32),
            pltpu.VMEM((1, H), jnp.float32),
        ],
        compiler_params=pltpu.CompilerParams(
            dimension_semantics=("arbitrary",),
        ),
    )(trial_feats, Wp, bp.reshape(1, H), W1, b1.reshape(1, H),
      W2, b2.reshape(1, O))
